# Initial kernel scaffold; baseline (speedup 1.0000x reference)
import jax
import jax.numpy as jnp
from jax.experimental import pallas as pl


def kernel(features, roi):
    raise NotImplementedError("write your pallas kernel here")


# speedup vs baseline: 20.4500x; 20.4500x over previous
import jax
import jax.numpy as jnp
from jax import lax
from jax.experimental import pallas as pl
from jax.experimental.pallas import tpu as pltpu

_B, _R = 4, 1000
_NKEEP = 64
_H, _W, _C = 200, 200, 256
_TILE_R = 64
_TILE_C = 72
_POOL = 3


def _nms_body(roi_s_ref, roi_t_ref, out_ref, m_ref):
    f32 = jnp.float32
    xj = roi_t_ref[:, 0:1, :]
    yj = roi_t_ref[:, 1:2, :]
    wj = roi_t_ref[:, 2:3, :]
    hj = roi_t_ref[:, 3:4, :]
    jlane = lax.broadcasted_iota(jnp.int32, (_B, 8, _R), 2)

    def mk_chunk(ci, carry):
        c0 = ci * 8
        blk = roi_s_ref[:, pl.ds(c0, 8), :]
        xi = blk[:, :, 0:1]
        yi = blk[:, :, 1:2]
        wi = blk[:, :, 2:3]
        hi = blk[:, :, 3:4]
        x1 = jnp.maximum(xi, xj)
        y1 = jnp.maximum(yi, yj)
        x2 = jnp.minimum(xi + wi, xj + wj)
        y2 = jnp.minimum(yi + hi, yj + hj)
        inter = jnp.maximum(0.0, x2 - x1) * jnp.maximum(0.0, y2 - y1)
        union = wi * hi + wj * hj - inter
        iou = inter / union
        isub = c0 + lax.broadcasted_iota(jnp.int32, (_B, 8, _R), 1)
        m = jnp.where((iou > 0.4) & (jlane > isub), f32(1.0), f32(0.0))
        m_ref[:, pl.ds(c0, 8), :] = m
        return carry

    lax.fori_loop(0, _R // 8, mk_chunk, 0)

    jl2 = lax.broadcasted_iota(jnp.int32, (_B, _R), 1)

    def scan_body(p, sup):
        row = m_ref[:, pl.ds(p, 1), :][:, 0, :]
        piv = jnp.max(jnp.where(jl2 == p, sup, f32(0.0)), axis=1, keepdims=True)
        return jnp.maximum(sup, row * (1.0 - piv))

    sup = lax.fori_loop(0, _R - 1, scan_body, jnp.zeros((_B, _R), f32))

    valid = 1.0 - sup
    c = valid
    sh = 1
    while sh < _R:
        z = jnp.zeros((_B, sh), f32)
        c = c + jnp.concatenate([z, c[:, : _R - sh]], axis=1)
        sh *= 2
    rank0 = c - 1.0

    kio = lax.broadcasted_iota(jnp.int32, (_B, _NKEEP, _R), 1)
    rank0i = rank0.astype(jnp.int32)
    oh = jnp.where(
        (valid[:, None, :] > 0.5) & (rank0i[:, None, :] == kio),
        f32(1.0), f32(0.0))
    found = jnp.sum(oh, axis=2)

    comps = []
    for d in range(4):
        cj = roi_t_ref[:, d:d + 1, :]
        s = jnp.sum(oh * cj, axis=2)
        s = jnp.where(found > 0.5, s, cj[:, :, _R - 1])
        comps.append(s)
    bx, by, bw, bh = comps

    xmin = jnp.maximum(0.0, bx).astype(jnp.int32)
    ymin = jnp.maximum(0.0, by).astype(jnp.int32)
    xmax = jnp.minimum(float(_W), bx + bw).astype(jnp.int32)
    ymax = jnp.minimum(float(_H), by + bh).astype(jnp.int32)

    def clip_axis(mn, mx, size, ps):
        pad = ps - (mx - mn)
        fix_min = mn < pad // 2
        fix_max = size - mx < (1 + pad) // 2
        pos = pad > 0
        sym = pos & (~(fix_min | fix_max))
        omin = jnp.where(sym, mn - pad // 2, mn)
        omax = jnp.where(sym, mx + (1 + pad) // 2, mx)
        omin = jnp.where(pos & fix_min, 0, omin)
        omax = jnp.where(pos & fix_min, ps, omax)
        omin = jnp.where(pos & fix_max, size - ps, omin)
        omax = jnp.where(pos & fix_max, size, omax)
        return omin, omax

    xo0, xo1 = clip_axis(xmin, xmax, _W, _POOL)
    yo0, yo1 = clip_axis(ymin, ymax, _H, _POOL)
    out_ref[...] = jnp.stack([xo0, yo0, xo1 - xo0, yo1 - yo0], axis=-1)


def _pool_body(roi_smem, feat_hbm, out_ref, buf, sem):
    n = pl.program_id(0)
    neg = jnp.float32(-jnp.inf)

    def coords(m):
        base = m * 4
        return (m // _NKEEP, roi_smem[base], roi_smem[base + 1],
                roi_smem[base + 2], roi_smem[base + 3])

    def copy_for(m, slot):
        b, x, y, w, h = coords(m)
        yc = jnp.minimum(y, _H - _TILE_R)
        xc = jnp.minimum((x // 8) * 8, _W - _TILE_C)
        return pltpu.make_async_copy(
            feat_hbm.at[b, pl.ds(yc, _TILE_R), pl.ds(xc, _TILE_C), :],
            buf.at[slot], sem.at[slot])

    @pl.when(n == 0)
    def _():
        copy_for(0, 0).start()

    @pl.when(n + 1 < _B * _NKEEP)
    def _():
        copy_for(n + 1, (n + 1) % 2).start()

    copy_for(n, n % 2).wait()

    slot = n % 2
    b, x, y, w, h = coords(n)
    yc = jnp.minimum(y, _H - _TILE_R)
    xc = jnp.minimum((x // 8) * 8, _W - _TILE_C)
    dy = y - yc
    dx = x - xc
    hs = h // _POOL
    ws = w // _POOL
    rlo = (dy, dy + hs, dy + 2 * hs)
    rhi = (dy + hs, dy + 2 * hs, dy + h)
    clo = (dx, dx + ws, dx + 2 * ws)
    chi = (dx + ws, dx + 2 * ws, dx + w)

    def chunk_body(rc, acc):
        data = buf[slot, pl.ds(rc * 4, 4), :, :]
        r4 = rc * 4 + lax.broadcasted_iota(jnp.int32, (4, 1, 1), 0)
        outs = []
        for i in range(3):
            msk = (r4 >= rlo[i]) & (r4 < rhi[i])
            red = jnp.max(jnp.where(msk, data, neg), axis=0)
            outs.append(jnp.maximum(acc[i], red))
        return tuple(outs)

    acc0 = jnp.full((_TILE_C, _C), neg, jnp.float32)
    accs = lax.fori_loop(0, _TILE_R // 4, chunk_body, (acc0, acc0, acc0))

    cio = lax.broadcasted_iota(jnp.int32, (_TILE_C, 1), 0)
    for i in range(3):
        rows_i = accs[i]
        for j in range(3):
            cm = (cio >= clo[j]) & (cio < chi[j])
            out_ref[0, 0, i, j] = jnp.max(jnp.where(cm, rows_i, neg), axis=0)


def kernel(features, roi):
    roi_t = jnp.transpose(roi, (0, 2, 1))

    roi_clipped = pl.pallas_call(
        _nms_body,
        out_shape=jax.ShapeDtypeStruct((_B, _NKEEP, 4), jnp.int32),
        in_specs=[
            pl.BlockSpec(memory_space=pltpu.VMEM),
            pl.BlockSpec(memory_space=pltpu.VMEM),
        ],
        out_specs=pl.BlockSpec(memory_space=pltpu.VMEM),
        scratch_shapes=[pltpu.VMEM((_B, _R, _R), jnp.float32)],
    )(roi, roi_t)

    roi_flat = roi_clipped.reshape(-1)

    pooled = pl.pallas_call(
        _pool_body,
        grid=(_B * _NKEEP,),
        out_shape=jax.ShapeDtypeStruct((_B, _NKEEP, _POOL, _POOL, _C),
                                       jnp.float32),
        in_specs=[
            pl.BlockSpec(memory_space=pltpu.SMEM),
            pl.BlockSpec(memory_space=pl.ANY),
        ],
        out_specs=pl.BlockSpec(
            (1, 1, _POOL, _POOL, _C),
            lambda n: (n // _NKEEP, n % _NKEEP, 0, 0, 0)),
        scratch_shapes=[
            pltpu.VMEM((2, _TILE_R, _TILE_C, _C), jnp.float32),
            pltpu.SemaphoreType.DMA((2,)),
        ],
        compiler_params=pltpu.CompilerParams(
            dimension_semantics=("arbitrary",)),
    )(roi_flat, features)

    return pooled, roi_clipped


# speedup vs baseline: 27.7591x; 1.3574x over previous
import jax
import jax.numpy as jnp
from jax import lax
from jax.experimental import pallas as pl
from jax.experimental.pallas import tpu as pltpu

_B, _R = 4, 1000
_NKEEP = 64
_H, _W, _C = 200, 200, 256
_TILE_R = 64
_TILE_C = 72
_POOL = 3


def _nms_body(roi_s_ref, roi_t_ref, out_ref, m_ref):
    f32 = jnp.float32
    xj = roi_t_ref[:, 0:1, :]
    yj = roi_t_ref[:, 1:2, :]
    wj = roi_t_ref[:, 2:3, :]
    hj = roi_t_ref[:, 3:4, :]
    jlane = lax.broadcasted_iota(jnp.int32, (_B, 8, _R), 2)

    def mk_chunk(ci, carry):
        c0 = ci * 8
        blk = roi_s_ref[:, pl.ds(c0, 8), :]
        xi = blk[:, :, 0:1]
        yi = blk[:, :, 1:2]
        wi = blk[:, :, 2:3]
        hi = blk[:, :, 3:4]
        x1 = jnp.maximum(xi, xj)
        y1 = jnp.maximum(yi, yj)
        x2 = jnp.minimum(xi + wi, xj + wj)
        y2 = jnp.minimum(yi + hi, yj + hj)
        inter = jnp.maximum(0.0, x2 - x1) * jnp.maximum(0.0, y2 - y1)
        union = wi * hi + wj * hj - inter
        iou = inter / union
        isub = c0 + lax.broadcasted_iota(jnp.int32, (_B, 8, _R), 1)
        m = jnp.where((iou > 0.4) & (jlane > isub), f32(1.0), f32(0.0))
        m_ref[:, pl.ds(c0, 8), :] = m
        return carry

    lax.fori_loop(0, _R // 8, mk_chunk, 0)

    jl2 = lax.broadcasted_iota(jnp.int32, (_B, _R), 1)

    def scan_body(p, sup):
        row = m_ref[:, pl.ds(p, 1), :][:, 0, :]
        piv = jnp.max(jnp.where(jl2 == p, sup, f32(0.0)), axis=1, keepdims=True)
        return jnp.maximum(sup, row * (1.0 - piv))

    sup = lax.fori_loop(0, _R - 1, scan_body, jnp.zeros((_B, _R), f32))

    valid = 1.0 - sup
    c = valid
    sh = 1
    while sh < _R:
        z = jnp.zeros((_B, sh), f32)
        c = c + jnp.concatenate([z, c[:, : _R - sh]], axis=1)
        sh *= 2
    rank0 = c - 1.0

    kio = lax.broadcasted_iota(jnp.int32, (_B, _NKEEP, _R), 1)
    rank0i = rank0.astype(jnp.int32)
    oh = jnp.where(
        (valid[:, None, :] > 0.5) & (rank0i[:, None, :] == kio),
        f32(1.0), f32(0.0))
    found = jnp.sum(oh, axis=2)

    comps = []
    for d in range(4):
        cj = roi_t_ref[:, d:d + 1, :]
        s = jnp.sum(oh * cj, axis=2)
        s = jnp.where(found > 0.5, s, cj[:, :, _R - 1])
        comps.append(s)
    bx, by, bw, bh = comps

    xmin = jnp.maximum(0.0, bx).astype(jnp.int32)
    ymin = jnp.maximum(0.0, by).astype(jnp.int32)
    xmax = jnp.minimum(float(_W), bx + bw).astype(jnp.int32)
    ymax = jnp.minimum(float(_H), by + bh).astype(jnp.int32)

    def clip_axis(mn, mx, size, ps):
        pad = ps - (mx - mn)
        fix_min = mn < pad // 2
        fix_max = size - mx < (1 + pad) // 2
        pos = pad > 0
        sym = pos & (~(fix_min | fix_max))
        omin = jnp.where(sym, mn - pad // 2, mn)
        omax = jnp.where(sym, mx + (1 + pad) // 2, mx)
        omin = jnp.where(pos & fix_min, 0, omin)
        omax = jnp.where(pos & fix_min, ps, omax)
        omin = jnp.where(pos & fix_max, size - ps, omin)
        omax = jnp.where(pos & fix_max, size, omax)
        return omin, omax

    xo0, xo1 = clip_axis(xmin, xmax, _W, _POOL)
    yo0, yo1 = clip_axis(ymin, ymax, _H, _POOL)
    out_ref[...] = jnp.stack([xo0, yo0, xo1 - xo0, yo1 - yo0], axis=-1)


def _pool_body(roi_smem, feat_hbm, out_ref, buf, sem):
    n = pl.program_id(0)
    neg = jnp.float32(-jnp.inf)

    def coords(m):
        base = m * 4
        return (m // _NKEEP, roi_smem[base], roi_smem[base + 1],
                roi_smem[base + 2], roi_smem[base + 3])

    def dma_chunks(m, slot):
        b, x, y, w, h = coords(m)
        yc = jnp.minimum(y, _H - _TILE_R)
        xc = jnp.minimum((x // 8) * 8, _W - _TILE_C)
        dy = y - yc
        dx = x - xc
        out = []
        for k in range(_TILE_R // 16):
            for c in range(_TILE_C // 24):
                cond = ((16 * k < dy + h) & (16 * k + 16 > dy)
                        & (24 * c < dx + w) & (24 * c + 24 > dx))
                desc = pltpu.make_async_copy(
                    feat_hbm.at[b, pl.ds(yc + 16 * k, 16),
                                pl.ds(xc + 24 * c, 24), :],
                    buf.at[slot, pl.ds(16 * k, 16), pl.ds(24 * c, 24), :],
                    sem.at[slot])
                out.append((cond, desc))
        return out

    @pl.when(n == 0)
    def _():
        for cond, desc in dma_chunks(0, 0):
            @pl.when(cond)
            def _(desc=desc):
                desc.start()

    @pl.when(n + 1 < _B * _NKEEP)
    def _():
        for cond, desc in dma_chunks(n + 1, (n + 1) % 2):
            @pl.when(cond)
            def _(desc=desc):
                desc.start()

    for cond, desc in dma_chunks(n, n % 2):
        @pl.when(cond)
        def _(desc=desc):
            desc.wait()

    slot = n % 2
    b, x, y, w, h = coords(n)
    yc = jnp.minimum(y, _H - _TILE_R)
    xc = jnp.minimum((x // 8) * 8, _W - _TILE_C)
    dy = y - yc
    dx = x - xc
    hs = h // _POOL
    ws = w // _POOL
    rlo = (dy, dy + hs, dy + 2 * hs)
    rhi = (dy + hs, dy + 2 * hs, dy + h)
    clo = (dx, dx + ws, dx + 2 * ws)
    chi = (dx + ws, dx + 2 * ws, dx + w)

    def chunk_body(rc, acc):
        data = buf[slot, pl.ds(rc * 4, 4), :, :]
        r4 = rc * 4 + lax.broadcasted_iota(jnp.int32, (4, 1, 1), 0)
        outs = []
        for i in range(3):
            msk = (r4 >= rlo[i]) & (r4 < rhi[i])
            red = jnp.max(jnp.where(msk, data, neg), axis=0)
            outs.append(jnp.maximum(acc[i], red))
        return tuple(outs)

    acc0 = jnp.full((_TILE_C, _C), neg, jnp.float32)
    accs = lax.fori_loop(dy // 4, (dy + h + 3) // 4, chunk_body,
                         (acc0, acc0, acc0))

    cio = lax.broadcasted_iota(jnp.int32, (_TILE_C, 1), 0)
    for i in range(3):
        rows_i = accs[i]
        for j in range(3):
            cm = (cio >= clo[j]) & (cio < chi[j])
            out_ref[0, 0, i, j] = jnp.max(jnp.where(cm, rows_i, neg), axis=0)


def kernel(features, roi):
    roi_t = jnp.transpose(roi, (0, 2, 1))

    roi_clipped = pl.pallas_call(
        _nms_body,
        out_shape=jax.ShapeDtypeStruct((_B, _NKEEP, 4), jnp.int32),
        in_specs=[
            pl.BlockSpec(memory_space=pltpu.VMEM),
            pl.BlockSpec(memory_space=pltpu.VMEM),
        ],
        out_specs=pl.BlockSpec(memory_space=pltpu.VMEM),
        scratch_shapes=[pltpu.VMEM((_B, _R, _R), jnp.float32)],
    )(roi, roi_t)

    roi_flat = roi_clipped.reshape(-1)

    pooled = pl.pallas_call(
        _pool_body,
        grid=(_B * _NKEEP,),
        out_shape=jax.ShapeDtypeStruct((_B, _NKEEP, _POOL, _POOL, _C),
                                       jnp.float32),
        in_specs=[
            pl.BlockSpec(memory_space=pltpu.SMEM),
            pl.BlockSpec(memory_space=pl.ANY),
        ],
        out_specs=pl.BlockSpec(
            (1, 1, _POOL, _POOL, _C),
            lambda n: (n // _NKEEP, n % _NKEEP, 0, 0, 0)),
        scratch_shapes=[
            pltpu.VMEM((2, _TILE_R, _TILE_C, _C), jnp.float32),
            pltpu.SemaphoreType.DMA((2,)),
        ],
        compiler_params=pltpu.CompilerParams(
            dimension_semantics=("arbitrary",)),
    )(roi_flat, features)

    return pooled, roi_clipped


# speedup vs baseline: 34.0312x; 1.2259x over previous
import jax
import jax.numpy as jnp
from jax import lax
from jax.experimental import pallas as pl
from jax.experimental.pallas import tpu as pltpu

_B, _R = 4, 1000
_NKEEP = 64
_H, _W, _C = 200, 200, 256
_TILE_R = 64
_TILE_C = 72
_POOL = 3


def _nms_body(roi_s_ref, roi_t_ref, out_ref, m_ref):
    f32 = jnp.float32
    xj = roi_t_ref[:, 0:1, :]
    yj = roi_t_ref[:, 1:2, :]
    wj = roi_t_ref[:, 2:3, :]
    hj = roi_t_ref[:, 3:4, :]
    jlane = lax.broadcasted_iota(jnp.int32, (_B, 8, _R), 2)

    def mk_chunk(ci, carry):
        c0 = ci * 8
        blk = roi_s_ref[:, pl.ds(c0, 8), :]
        xi = blk[:, :, 0:1]
        yi = blk[:, :, 1:2]
        wi = blk[:, :, 2:3]
        hi = blk[:, :, 3:4]
        x1 = jnp.maximum(xi, xj)
        y1 = jnp.maximum(yi, yj)
        x2 = jnp.minimum(xi + wi, xj + wj)
        y2 = jnp.minimum(yi + hi, yj + hj)
        inter = jnp.maximum(0.0, x2 - x1) * jnp.maximum(0.0, y2 - y1)
        union = wi * hi + wj * hj - inter
        iou = inter / union
        isub = c0 + lax.broadcasted_iota(jnp.int32, (_B, 8, _R), 1)
        m = jnp.where((iou > 0.4) & (jlane > isub), f32(1.0), f32(0.0))
        m_ref[:, pl.ds(c0, 8), :] = m
        return carry

    lax.fori_loop(0, _R // 8, mk_chunk, 0)

    jl2 = lax.broadcasted_iota(jnp.int32, (_B, _R), 1)

    def scan_body(p, sup):
        row = m_ref[:, pl.ds(p, 1), :][:, 0, :]
        piv = jnp.max(jnp.where(jl2 == p, sup, f32(0.0)), axis=1, keepdims=True)
        return jnp.maximum(sup, row * (1.0 - piv))

    sup = lax.fori_loop(0, _R - 1, scan_body, jnp.zeros((_B, _R), f32))

    valid = 1.0 - sup
    c = valid
    sh = 1
    while sh < _R:
        z = jnp.zeros((_B, sh), f32)
        c = c + jnp.concatenate([z, c[:, : _R - sh]], axis=1)
        sh *= 2
    rank0 = c - 1.0

    kio = lax.broadcasted_iota(jnp.int32, (_B, _NKEEP, _R), 1)
    rank0i = rank0.astype(jnp.int32)
    oh = jnp.where(
        (valid[:, None, :] > 0.5) & (rank0i[:, None, :] == kio),
        f32(1.0), f32(0.0))
    found = jnp.sum(oh, axis=2)

    comps = []
    for d in range(4):
        cj = roi_t_ref[:, d:d + 1, :]
        s = jnp.sum(oh * cj, axis=2)
        s = jnp.where(found > 0.5, s, cj[:, :, _R - 1])
        comps.append(s)
    bx, by, bw, bh = comps

    xmin = jnp.maximum(0.0, bx).astype(jnp.int32)
    ymin = jnp.maximum(0.0, by).astype(jnp.int32)
    xmax = jnp.minimum(float(_W), bx + bw).astype(jnp.int32)
    ymax = jnp.minimum(float(_H), by + bh).astype(jnp.int32)

    def clip_axis(mn, mx, size, ps):
        pad = ps - (mx - mn)
        fix_min = mn < pad // 2
        fix_max = size - mx < (1 + pad) // 2
        pos = pad > 0
        sym = pos & (~(fix_min | fix_max))
        omin = jnp.where(sym, mn - pad // 2, mn)
        omax = jnp.where(sym, mx + (1 + pad) // 2, mx)
        omin = jnp.where(pos & fix_min, 0, omin)
        omax = jnp.where(pos & fix_min, ps, omax)
        omin = jnp.where(pos & fix_max, size - ps, omin)
        omax = jnp.where(pos & fix_max, size, omax)
        return omin, omax

    xo0, xo1 = clip_axis(xmin, xmax, _W, _POOL)
    yo0, yo1 = clip_axis(ymin, ymax, _H, _POOL)
    out_ref[...] = jnp.stack([xo0, yo0, xo1 - xo0, yo1 - yo0], axis=-1)


def _pool_body(roi_smem, feat_hbm, out_ref, buf, sem):
    n = pl.program_id(0)
    neg = jnp.float32(-jnp.inf)

    def coords(m):
        base = m * 4
        return (m // _NKEEP, roi_smem[base], roi_smem[base + 1],
                roi_smem[base + 2], roi_smem[base + 3])

    def dma_chunks(m, slot):
        b, x, y, w, h = coords(m)
        yc = jnp.minimum(y, _H - _TILE_R)
        xc = jnp.minimum((x // 8) * 8, _W - _TILE_C)
        dy = y - yc
        dx = x - xc
        out = []
        for k in range(_TILE_R // 16):
            for c in range(_TILE_C // 24):
                cond = ((16 * k < dy + h) & (16 * k + 16 > dy)
                        & (24 * c < dx + w) & (24 * c + 24 > dx))
                desc = pltpu.make_async_copy(
                    feat_hbm.at[b, pl.ds(yc + 16 * k, 16),
                                pl.ds(xc + 24 * c, 24), :],
                    buf.at[slot, pl.ds(16 * k, 16), pl.ds(24 * c, 24), :],
                    sem.at[slot])
                out.append((cond, desc))
        return out

    @pl.when(n == 0)
    def _():
        for cond, desc in dma_chunks(0, 0):
            @pl.when(cond)
            def _(desc=desc):
                desc.start()

    @pl.when(n + 1 < _B * _NKEEP)
    def _():
        for cond, desc in dma_chunks(n + 1, (n + 1) % 2):
            @pl.when(cond)
            def _(desc=desc):
                desc.start()

    for cond, desc in dma_chunks(n, n % 2):
        @pl.when(cond)
        def _(desc=desc):
            desc.wait()

    slot = n % 2
    b, x, y, w, h = coords(n)
    yc = jnp.minimum(y, _H - _TILE_R)
    xc = jnp.minimum((x // 8) * 8, _W - _TILE_C)
    dy = y - yc
    dx = x - xc
    hs = h // _POOL
    ws = w // _POOL
    rlo = (dy, dy + hs, dy + 2 * hs)
    rhi = (dy + hs, dy + 2 * hs, dy + h)
    clo = (dx, dx + ws, dx + 2 * ws)
    chi = (dx + ws, dx + 2 * ws, dx + w)

    acc0 = jnp.full((_TILE_C, _C), neg, jnp.float32)

    def row_max(r, a):
        return jnp.maximum(a, buf[slot, r])

    cio = lax.broadcasted_iota(jnp.int32, (_TILE_C, 1), 0)
    for i in range(3):
        rows_i = lax.fori_loop(rlo[i], rhi[i], row_max, acc0)
        for j in range(3):
            cm = (cio >= clo[j]) & (cio < chi[j])
            out_ref[0, 0, i, j] = jnp.max(jnp.where(cm, rows_i, neg), axis=0)


def kernel(features, roi):
    roi_t = jnp.transpose(roi, (0, 2, 1))

    roi_clipped = pl.pallas_call(
        _nms_body,
        out_shape=jax.ShapeDtypeStruct((_B, _NKEEP, 4), jnp.int32),
        in_specs=[
            pl.BlockSpec(memory_space=pltpu.VMEM),
            pl.BlockSpec(memory_space=pltpu.VMEM),
        ],
        out_specs=pl.BlockSpec(memory_space=pltpu.VMEM),
        scratch_shapes=[pltpu.VMEM((_B, _R, _R), jnp.float32)],
    )(roi, roi_t)

    roi_flat = roi_clipped.reshape(-1)

    pooled = pl.pallas_call(
        _pool_body,
        grid=(_B * _NKEEP,),
        out_shape=jax.ShapeDtypeStruct((_B, _NKEEP, _POOL, _POOL, _C),
                                       jnp.float32),
        in_specs=[
            pl.BlockSpec(memory_space=pltpu.SMEM),
            pl.BlockSpec(memory_space=pl.ANY),
        ],
        out_specs=pl.BlockSpec(
            (1, 1, _POOL, _POOL, _C),
            lambda n: (n // _NKEEP, n % _NKEEP, 0, 0, 0)),
        scratch_shapes=[
            pltpu.VMEM((2, _TILE_R, _TILE_C, _C), jnp.float32),
            pltpu.SemaphoreType.DMA((2,)),
        ],
        compiler_params=pltpu.CompilerParams(
            dimension_semantics=("arbitrary",)),
    )(roi_flat, features)

    return pooled, roi_clipped
